# baseline (speedup 1.0000x reference)
import functools

import jax
import jax.numpy as jnp
from jax import lax
from jax.experimental import pallas as pl
from jax.experimental.pallas import tpu as pltpu
from jax.experimental.pallas import tpu_sc as plsc

N = 10000
E = 320000
NC = 2
NS = 16
EB = 128
BPW = 80
HB = 40
EPAD = NC * NS * BPW * EB
NPAD = 10240
RPT = NPAD // NS


_GDN = lax.GatherDimensionNumbers(
    offset_dims=(), collapsed_slice_dims=(0,), start_index_map=(0,))


def _lane_broadcast(vec, lane):
    idx = jnp.full((16, 1), lane, jnp.int32)
    return lax.gather(vec, idx, _GDN, slice_sizes=(1,),
                      mode=lax.GatherScatterMode.PROMISE_IN_BOUNDS)


def _make_spmm(W):
    FC = W // 16

    mesh = plsc.VectorSubcoreMesh(core_axis_name="c", subcore_axis_name="s")

    scratch = [
        pltpu.VMEM((HB, EB), jnp.int32),
        pltpu.VMEM((HB, EB), jnp.int32),
        pltpu.VMEM((HB, EB), jnp.float32),
        pltpu.VMEM_SHARED((NPAD, W), jnp.float32),
        pltpu.VMEM((EB, W), jnp.float32),
        pltpu.VMEM((EB, W), jnp.float32),
        pltpu.SemaphoreType.DMA,
        pltpu.SemaphoreType.DMA,
        pltpu.SemaphoreType.DMA,
        pltpu.SemaphoreType.DMA,
    ]

    @functools.partial(
        pl.kernel,
        out_type=jax.ShapeDtypeStruct((NC, NPAD, W), jnp.float32),
        mesh=mesh,
        scratch_types=scratch,
    )
    def spmm(h_hbm, src_hbm, dst_hbm, ew_hbm, zero_hbm, out_hbm,
             srcv, dstv, wv, acc, r0, r1, g0, g1, s0, s1):
        rows = [r0, r1]
        gsem = [g0, g1]
        ssem = [s0, s1]
        cid = lax.axis_index("c")
        sid = lax.axis_index("s")
        wid = cid * NS + sid

        pltpu.sync_copy(zero_hbm, acc.at[pl.ds(sid * RPT, RPT)])
        plsc.subcore_barrier()

        def start_gather(j, b):
            pltpu.async_copy(h_hbm.at[srcv.at[j]], rows[b], gsem[b])

        def wait_gather(j, b):
            pltpu.make_async_copy(h_hbm.at[srcv.at[j]], rows[b], gsem[b]).wait()

        def start_scatter(j, b):
            pltpu.async_copy(rows[b], acc.at[dstv.at[j]], ssem[b], add=True)

        def wait_scatter(j, b):
            pltpu.make_async_copy(rows[b], acc.at[dstv.at[j]], ssem[b]).wait()

        def scale(j, b):
            def grp(g, c2):
                wg = wv[j, pl.ds(g * 16, 16)]
                for i in range(16):
                    ws = _lane_broadcast(wg, i)
                    e = g * 16 + i
                    for f in range(FC):
                        sl = pl.ds(f * 16, 16)
                        rows[b][e, sl] = rows[b][e, sl] * ws
                return c2

            lax.fori_loop(0, EB // 16, grp, 0)

        for h in range(BPW // HB):
            cbase = wid * BPW + h * HB
            pltpu.sync_copy(src_hbm.at[pl.ds(cbase, HB)], srcv)
            pltpu.sync_copy(dst_hbm.at[pl.ds(cbase, HB)], dstv)
            pltpu.sync_copy(ew_hbm.at[pl.ds(cbase, HB)], wv)

            start_gather(0, 0)

            def outer(jj, carry):
                for b in range(2):
                    j = jj * 2 + b
                    if b == 0:
                        @pl.when(jj >= 1)
                        def _():
                            wait_scatter(j - 1, 1)
                        start_gather(j + 1, 1)
                    else:
                        wait_scatter(j - 1, 0)

                        @pl.when(jj < HB // 2 - 1)
                        def _():
                            start_gather(j + 1, 0)
                    wait_gather(j, b)
                    scale(j, b)
                    start_scatter(j, b)
                return carry

            lax.fori_loop(0, HB // 2, outer, 0)
            wait_scatter(HB - 1, 1)

        plsc.subcore_barrier()
        rr = sid * RPT
        pltpu.sync_copy(acc.at[pl.ds(rr, RPT)], out_hbm.at[cid, pl.ds(rr, RPT)])

    return spmm


_spmm128 = _make_spmm(128)

_BM = 1000


def _mm_body(x_ref, w_ref, o_ref):
    o_ref[...] = jnp.dot(x_ref[...], w_ref[...],
                         preferred_element_type=jnp.float32)


def _matmul_tc(x, w):
    m, k = x.shape
    n = w.shape[1]
    return pl.pallas_call(
        _mm_body,
        grid=(m // _BM,),
        in_specs=[pl.BlockSpec((_BM, k), lambda i: (i, 0)),
                  pl.BlockSpec((k, n), lambda i: (0, 0))],
        out_specs=pl.BlockSpec((_BM, n), lambda i: (i, 0)),
        out_shape=jax.ShapeDtypeStruct((m, n), jnp.float32),
    )(x, w)


def _mid_body(p0_ref, p1_ref, b_ref, w_ref, o_ref):
    h = jnp.maximum(p0_ref[...] + p1_ref[...] + b_ref[...], 0.0)
    o_ref[...] = jnp.dot(h, w_ref[...], preferred_element_type=jnp.float32)


def _mid_tc(p0, p1, b1, w2):
    m, k = p0.shape
    n = w2.shape[1]
    return pl.pallas_call(
        _mid_body,
        grid=(m // _BM,),
        in_specs=[pl.BlockSpec((_BM, k), lambda i: (i, 0)),
                  pl.BlockSpec((_BM, k), lambda i: (i, 0)),
                  pl.BlockSpec((1, k), lambda i: (0, 0)),
                  pl.BlockSpec((k, n), lambda i: (0, 0))],
        out_specs=pl.BlockSpec((_BM, n), lambda i: (i, 0)),
        out_shape=jax.ShapeDtypeStruct((m, n), jnp.float32),
    )(p0, p1, b1, w2)


def _final_body(q0_ref, q1_ref, b_ref, o_ref):
    z = q0_ref[...] + q1_ref[...] + b_ref[...]
    z = z - jnp.max(z, axis=1, keepdims=True)
    o_ref[...] = z - jnp.log(jnp.sum(jnp.exp(z), axis=1, keepdims=True))


def _final_tc(q0, q1, b2):
    m, n = q0.shape
    return pl.pallas_call(
        _final_body,
        grid=(m // _BM,),
        in_specs=[pl.BlockSpec((_BM, n), lambda i: (i, 0)),
                  pl.BlockSpec((_BM, n), lambda i: (i, 0)),
                  pl.BlockSpec((1, n), lambda i: (0, 0))],
        out_specs=pl.BlockSpec((_BM, n), lambda i: (i, 0)),
        out_shape=jax.ShapeDtypeStruct((m, n), jnp.float32),
    )(q0, q1, b2)


def kernel(x, edge_index, edge_weight, W1, b1, W2, b2):
    pad = EPAD - edge_index.shape[1]
    src = jnp.concatenate([edge_index[0],
                           jnp.zeros((pad,), jnp.int32)]).reshape(-1, EB)
    dst = jnp.concatenate([edge_index[1],
                           jnp.zeros((pad,), jnp.int32)]).reshape(-1, EB)
    ew = jnp.concatenate([edge_weight,
                          jnp.zeros((pad,), jnp.float32)]).reshape(-1, EB)
    zeros = jnp.zeros((RPT, 128), jnp.float32)
    h1 = _matmul_tc(x, W1)
    p = _spmm128(h1, src, dst, ew, zeros)
    w2p = jnp.concatenate([W2, jnp.zeros((W2.shape[0], 128 - W2.shape[1]),
                                         jnp.float32)], axis=1)
    h2 = _mid_tc(p[0, :N], p[1, :N], b1.reshape(1, -1), w2p)
    q = _spmm128(h2, src, dst, ew, zeros)
    ncls = W2.shape[1]
    return _final_tc(q[0, :N, :ncls], q[1, :N, :ncls], b2.reshape(1, -1))


# speedup vs baseline: 2.8657x; 2.8657x over previous
import functools

import jax
import jax.numpy as jnp
from jax import lax
from jax.experimental import pallas as pl
from jax.experimental.pallas import tpu as pltpu
from jax.experimental.pallas import tpu_sc as plsc

N = 10000
E = 320000
NC = 2
NS = 16
EB = 128
BPW = 80
HB = 40
EPAD = NC * NS * BPW * EB
NPAD = 10240
RPT = NPAD // NS


_GDN = lax.GatherDimensionNumbers(
    offset_dims=(), collapsed_slice_dims=(0,), start_index_map=(0,))


def _lane_broadcast(vec, lane):
    idx = jnp.full((16, 1), lane, jnp.int32)
    return lax.gather(vec, idx, _GDN, slice_sizes=(1,),
                      mode=lax.GatherScatterMode.PROMISE_IN_BOUNDS)


def _make_spmm(W):
    FC = W // 16

    mesh = plsc.VectorSubcoreMesh(core_axis_name="c", subcore_axis_name="s")

    scratch = [
        pltpu.VMEM((HB, EB), jnp.int32),
        pltpu.VMEM((HB, EB), jnp.int32),
        pltpu.VMEM((HB, EB), jnp.float32),
        pltpu.VMEM_SHARED((NPAD, W), jnp.float32),
        pltpu.VMEM((EB, W), jnp.float32),
        pltpu.VMEM((EB, W), jnp.float32),
        pltpu.SemaphoreType.DMA,
        pltpu.SemaphoreType.DMA,
        pltpu.SemaphoreType.DMA,
        pltpu.SemaphoreType.DMA,
    ]

    @functools.partial(
        pl.kernel,
        out_type=jax.ShapeDtypeStruct((NC, NPAD, W), jnp.float32),
        mesh=mesh,
        scratch_types=scratch,
    )
    def spmm(h_hbm, src_hbm, dst_hbm, ew_hbm, zero_hbm, out_hbm,
             srcv, dstv, wv, acc, r0, r1, g0, g1, s0, s1):
        rows = [r0, r1]
        gsem = [g0, g1]
        ssem = [s0, s1]
        cid = lax.axis_index("c")
        sid = lax.axis_index("s")
        wid = cid * NS + sid

        pltpu.sync_copy(zero_hbm, acc.at[pl.ds(sid * RPT, RPT)])
        plsc.subcore_barrier()

        def start_gather(j, b):
            pltpu.async_copy(h_hbm.at[srcv.at[j]], rows[b], gsem[b])

        def wait_gather(j, b):
            pltpu.make_async_copy(h_hbm.at[srcv.at[j]], rows[b], gsem[b]).wait()

        def start_scatter(j, b):
            pltpu.async_copy(rows[b], acc.at[dstv.at[j]], ssem[b], add=True)

        def wait_scatter(j, b):
            pltpu.make_async_copy(rows[b], acc.at[dstv.at[j]], ssem[b]).wait()

        def scale(j, b):
            def grp(g, c2):
                wg = wv[j, pl.ds(g * 16, 16)]
                for i in range(16):
                    ws = _lane_broadcast(wg, i)
                    e = g * 16 + i
                    for f in range(FC):
                        sl = pl.ds(f * 16, 16)
                        rows[b][e, sl] = rows[b][e, sl] * ws
                return c2

            lax.fori_loop(0, EB // 16, grp, 0)

        for h in range(BPW // HB):
            cbase = wid * BPW + h * HB
            pltpu.sync_copy(src_hbm.at[pl.ds(cbase, HB)], srcv)
            pltpu.sync_copy(dst_hbm.at[pl.ds(cbase, HB)], dstv)
            pltpu.sync_copy(ew_hbm.at[pl.ds(cbase, HB)], wv)

            start_gather(0, 0)

            def outer(jj, carry):
                for b in range(2):
                    j = jj * 2 + b
                    if b == 0:
                        @pl.when(jj >= 1)
                        def _():
                            wait_scatter(j - 1, 1)
                        start_gather(j + 1, 1)
                    else:
                        wait_scatter(j - 1, 0)

                        @pl.when(jj < HB // 2 - 1)
                        def _():
                            start_gather(j + 1, 0)
                    wait_gather(j, b)
                    scale(j, b)
                    start_scatter(j, b)
                return carry

            lax.fori_loop(0, HB // 2, outer, 0)
            wait_scatter(HB - 1, 1)

        plsc.subcore_barrier()
        rr = sid * RPT
        pltpu.sync_copy(acc.at[pl.ds(rr, RPT)], out_hbm.at[cid, pl.ds(rr, RPT)])

    return spmm


_spmm128 = _make_spmm(128)

_BM = 1000


def _mm_body(x_ref, w_ref, o_ref):
    o_ref[...] = jnp.dot(x_ref[...], w_ref[...],
                         preferred_element_type=jnp.float32)


def _matmul_tc(x, w):
    m, k = x.shape
    n = w.shape[1]
    return pl.pallas_call(
        _mm_body,
        grid=(m // _BM,),
        in_specs=[pl.BlockSpec((_BM, k), lambda i: (i, 0)),
                  pl.BlockSpec((k, n), lambda i: (0, 0))],
        out_specs=pl.BlockSpec((_BM, n), lambda i: (i, 0)),
        out_shape=jax.ShapeDtypeStruct((m, n), jnp.float32),
    )(x, w)


def _mid_body(p0_ref, p1_ref, b_ref, w_ref, o_ref):
    h = jnp.maximum(p0_ref[...] + p1_ref[...] + b_ref[...], 0.0)
    o_ref[...] = jnp.dot(h, w_ref[...], preferred_element_type=jnp.float32)


def _mid_tc(p0, p1, b1, w2):
    m, k = p0.shape
    n = w2.shape[1]
    return pl.pallas_call(
        _mid_body,
        grid=(m // _BM,),
        in_specs=[pl.BlockSpec((_BM, k), lambda i: (i, 0)),
                  pl.BlockSpec((_BM, k), lambda i: (i, 0)),
                  pl.BlockSpec((1, k), lambda i: (0, 0)),
                  pl.BlockSpec((k, n), lambda i: (0, 0))],
        out_specs=pl.BlockSpec((_BM, n), lambda i: (i, 0)),
        out_shape=jax.ShapeDtypeStruct((m, n), jnp.float32),
    )(p0, p1, b1, w2)


def _final_body(q0_ref, q1_ref, b_ref, o_ref):
    z = q0_ref[...] + q1_ref[...] + b_ref[...]
    z = z - jnp.max(z, axis=1, keepdims=True)
    o_ref[...] = z - jnp.log(jnp.sum(jnp.exp(z), axis=1, keepdims=True))


def _final_tc(q0, q1, b2):
    m, n = q0.shape
    return pl.pallas_call(
        _final_body,
        grid=(m // _BM,),
        in_specs=[pl.BlockSpec((_BM, n), lambda i: (i, 0)),
                  pl.BlockSpec((_BM, n), lambda i: (i, 0)),
                  pl.BlockSpec((1, n), lambda i: (0, 0))],
        out_specs=pl.BlockSpec((_BM, n), lambda i: (i, 0)),
        out_shape=jax.ShapeDtypeStruct((m, n), jnp.float32),
    )(q0, q1, b2)


def kernel(x, edge_index, edge_weight, W1, b1, W2, b2):
    pad = EPAD - edge_index.shape[1]
    spread = jnp.arange(pad, dtype=jnp.int32) % N
    src = jnp.concatenate([edge_index[0], spread]).reshape(-1, EB)
    dst = jnp.concatenate([edge_index[1], spread]).reshape(-1, EB)
    ew = jnp.concatenate([edge_weight,
                          jnp.zeros((pad,), jnp.float32)]).reshape(-1, EB)
    zeros = jnp.zeros((RPT, 128), jnp.float32)
    h1 = _matmul_tc(x, W1)
    p = _spmm128(h1, src, dst, ew, zeros)
    w2p = jnp.concatenate([W2, jnp.zeros((W2.shape[0], 128 - W2.shape[1]),
                                         jnp.float32)], axis=1)
    h2 = _mid_tc(p[0, :N], p[1, :N], b1.reshape(1, -1), w2p)
    q = _spmm128(h2, src, dst, ew, zeros)
    ncls = W2.shape[1]
    return _final_tc(q[0, :N, :ncls], q[1, :N, :ncls], b2.reshape(1, -1))
